# Initial kernel scaffold; baseline (speedup 1.0000x reference)
import jax
import jax.numpy as jnp
from jax.experimental import pallas as pl


def kernel(node_features, senders, receivers, edge_features, We1, be1, We2, be2, ge, bbe, Wn1, bn1, Wn2, bn2, gn, bbn):
    raise NotImplementedError("write your pallas kernel here")


# speedup vs baseline: 3.6764x; 3.6764x over previous
import functools

import jax
import jax.numpy as jnp
from jax import lax
from jax.experimental import pallas as pl
from jax.experimental.pallas import tpu as pltpu
from jax.experimental.pallas import tpu_sc as plsc

N = 10000
E = 320000
D = 128
H = 128

NC = 2
NS = 16
NW = NC * NS
EPW = E // NW

GC = 400
GSUB = 80
SC_CHUNK = 80

@functools.lru_cache(maxsize=None)
def _mesh():
    return plsc.VectorSubcoreMesh(
        core_axis_name="c", subcore_axis_name="s", num_cores=NC, num_subcores=NS)


def _pre_body(nf_ref, we1a_ref, we1b_ref, wn1a_ref, ps_ref, pr_ref, qn_ref):
    nf = nf_ref[...]
    ps_ref[...] = jnp.dot(nf, we1a_ref[...], preferred_element_type=jnp.float32)
    pr_ref[...] = jnp.dot(nf, we1b_ref[...], preferred_element_type=jnp.float32)
    qn_ref[...] = jnp.dot(nf, wn1a_ref[...], preferred_element_type=jnp.float32)


def _pre(nf, we1a, we1b, wn1a):
    out = jax.ShapeDtypeStruct((N, D), jnp.float32)
    return pl.pallas_call(
        _pre_body,
        out_shape=(out, out, out),
    )(nf, we1a, we1b, wn1a)


def _gather_body(ps_hbm, pr_hbm, s_hbm, r_hbm, out_hbm,
                 sidx, ridx, abuf, bbuf, sema, semb):
    wid = lax.axis_index("c") * NS + lax.axis_index("s")
    base0 = wid * EPW

    def chunk(k, carry):
        base = base0 + k * GC
        pltpu.sync_copy(s_hbm.at[pl.ds(base, GC)], sidx)
        pltpu.sync_copy(r_hbm.at[pl.ds(base, GC)], ridx)
        cps = []
        for j in range(GC // GSUB):
            sl = pl.ds(j * GSUB, GSUB)
            cps.append(pltpu.async_copy(ps_hbm.at[sidx.at[sl]], abuf.at[sl], sema))
            cps.append(pltpu.async_copy(pr_hbm.at[ridx.at[sl]], bbuf.at[sl], semb))
        for cp in cps:
            cp.wait()

        def addrow(i, c):
            for j in range(D // 16):
                sl16 = pl.ds(j * 16, 16)
                abuf[i, sl16] = abuf[i, sl16] + bbuf[i, sl16]
            return c

        lax.fori_loop(0, GC, addrow, 0)
        pltpu.sync_copy(abuf, out_hbm.at[pl.ds(base, GC)])
        return carry

    lax.fori_loop(0, EPW // GC, chunk, 0)


@functools.lru_cache(maxsize=None)
def _gather_kernel():
    return pl.kernel(
        _gather_body,
        out_type=jax.ShapeDtypeStruct((E, D), jnp.float32),
        mesh=_mesh(),
        scratch_types=[
            pltpu.VMEM((GC,), jnp.int32),
            pltpu.VMEM((GC,), jnp.int32),
            pltpu.VMEM((GC, D), jnp.float32),
            pltpu.VMEM((GC, D), jnp.float32),
            pltpu.SemaphoreType.DMA,
            pltpu.SemaphoreType.DMA,
        ],
    )


def _edge_body(g_ref, ef_ref, we1c_ref, be1_ref, we2_ref, be2_ref,
               ge_ref, bbe_ref, oe_ref, ne_ref):
    ef = ef_ref[...]
    x = g_ref[...] + jnp.dot(ef, we1c_ref[...],
                             preferred_element_type=jnp.float32) + be1_ref[...]
    h = jnp.maximum(x, 0.0)
    h2 = jnp.dot(h, we2_ref[...], preferred_element_type=jnp.float32) + be2_ref[...]
    mu = jnp.mean(h2, axis=-1, keepdims=True)
    var = jnp.mean((h2 - mu) * (h2 - mu), axis=-1, keepdims=True)
    ne = (h2 - mu) * lax.rsqrt(var + 1e-5) * ge_ref[...] + bbe_ref[...]
    ne_ref[...] = ne
    oe_ref[...] = ef + ne


def _edge_mlp(g, ef, we1c, be1, we2, be2, ge, bbe):
    BE = 2000
    grid = E // BE
    row_spec = pl.BlockSpec((BE, D), lambda i: (i, 0))
    w_spec = pl.BlockSpec((D, H), lambda i: (0, 0))
    v_spec = pl.BlockSpec((1, D), lambda i: (0, 0))
    out = jax.ShapeDtypeStruct((E, D), jnp.float32)
    return pl.pallas_call(
        _edge_body,
        grid=(grid,),
        in_specs=[row_spec, row_spec, w_spec, v_spec, w_spec, v_spec, v_spec, v_spec],
        out_specs=(row_spec, row_spec),
        out_shape=(out, out),
    )(g, ef, we1c, be1, we2, be2, ge, bbe)


def _segsum_body(ne_hbm, r_hbm, zeros_hbm, out_hbm, ridx, buf, agg):
    cid = lax.axis_index("c")
    sid = lax.axis_index("s")

    @pl.when(sid == 0)
    def _():
        pltpu.sync_copy(zeros_hbm, agg)

    plsc.subcore_barrier()

    base0 = (cid * NS + sid) * EPW

    def chunk(k, carry):
        base = base0 + k * SC_CHUNK
        pltpu.sync_copy(r_hbm.at[pl.ds(base, SC_CHUNK)], ridx)
        pltpu.sync_copy(ne_hbm.at[pl.ds(base, SC_CHUNK)], buf)
        pltpu.sync_copy(buf, agg.at[ridx], add=True)
        return carry

    lax.fori_loop(0, EPW // SC_CHUNK, chunk, 0)

    plsc.subcore_barrier()

    rows = 624
    sl = pl.ds(sid * rows, rows)
    pltpu.sync_copy(agg.at[sl], out_hbm.at[cid].at[sl])

    @pl.when(sid == 0)
    def _():
        tail = pl.ds(NS * rows, N - NS * rows)
        pltpu.sync_copy(agg.at[tail], out_hbm.at[cid].at[tail])


@functools.lru_cache(maxsize=None)
def _segsum_kernel():
    return pl.kernel(
        _segsum_body,
        out_type=jax.ShapeDtypeStruct((NC, N, D), jnp.float32),
        mesh=_mesh(),
        scratch_types=[
            pltpu.VMEM((SC_CHUNK,), jnp.int32),
            pltpu.VMEM((SC_CHUNK, D), jnp.float32),
            pltpu.VMEM_SHARED((N, D), jnp.float32),
        ],
    )


def _node_body(nf_ref, qn_ref, p0_ref, p1_ref, wn1b_ref, bn1_ref,
               wn2_ref, bn2_ref, gn_ref, bbn_ref, on_ref):
    agg = p0_ref[...] + p1_ref[...]
    x = qn_ref[...] + jnp.dot(agg, wn1b_ref[...],
                              preferred_element_type=jnp.float32) + bn1_ref[...]
    h = jnp.maximum(x, 0.0)
    h2 = jnp.dot(h, wn2_ref[...], preferred_element_type=jnp.float32) + bn2_ref[...]
    mu = jnp.mean(h2, axis=-1, keepdims=True)
    var = jnp.mean((h2 - mu) * (h2 - mu), axis=-1, keepdims=True)
    nn = (h2 - mu) * lax.rsqrt(var + 1e-5) * gn_ref[...] + bbn_ref[...]
    on_ref[...] = nf_ref[...] + nn


def _node_mlp(nf, qn, p0, p1, wn1b, bn1, wn2, bn2, gn, bbn):
    BN = 2000
    grid = N // BN
    row_spec = pl.BlockSpec((BN, D), lambda i: (i, 0))
    w_spec = pl.BlockSpec((D, H), lambda i: (0, 0))
    v_spec = pl.BlockSpec((1, D), lambda i: (0, 0))
    return pl.pallas_call(
        _node_body,
        grid=(grid,),
        in_specs=[row_spec, row_spec, row_spec, row_spec,
                  w_spec, v_spec, w_spec, v_spec, v_spec, v_spec],
        out_specs=row_spec,
        out_shape=jax.ShapeDtypeStruct((N, D), jnp.float32),
    )(nf, qn, p0, p1, wn1b, bn1, wn2, bn2, gn, bbn)


def kernel(node_features, senders, receivers, edge_features,
           We1, be1, We2, be2, ge, bbe, Wn1, bn1, Wn2, bn2, gn, bbn):
    senders = senders.astype(jnp.int32)
    receivers = receivers.astype(jnp.int32)

    we1a, we1b, we1c = We1[0:D], We1[D:2 * D], We1[2 * D:3 * D]
    wn1a, wn1b = Wn1[0:D], Wn1[D:2 * D]

    ps, pr, qn = _pre(node_features, we1a, we1b, wn1a)
    g = _gather_kernel()(ps, pr, senders, receivers)
    out_edges, new_edge = _edge_mlp(
        g, edge_features, we1c,
        be1.reshape(1, D), We2, be2.reshape(1, D),
        ge.reshape(1, D), bbe.reshape(1, D))
    zeros = jnp.zeros((N, D), jnp.float32)
    parts = _segsum_kernel()(new_edge, receivers, zeros)
    out_nodes = _node_mlp(
        node_features, qn, parts[0], parts[1],
        wn1b, bn1.reshape(1, D), Wn2, bn2.reshape(1, D),
        gn.reshape(1, D), bbn.reshape(1, D))
    return (out_nodes, out_edges)


# speedup vs baseline: 3.7434x; 1.0182x over previous
import functools

import jax
import jax.numpy as jnp
from jax import lax
from jax.experimental import pallas as pl
from jax.experimental.pallas import tpu as pltpu
from jax.experimental.pallas import tpu_sc as plsc

N = 10000
E = 320000
D = 128
H = 128

NC = 2
NS = 16
NW = NC * NS
EPW = E // NW

GC = 80
NCHUNK = EPW // GC

@functools.lru_cache(maxsize=None)
def _mesh():
    return plsc.VectorSubcoreMesh(
        core_axis_name="c", subcore_axis_name="s", num_cores=NC, num_subcores=NS)


def _pre_body(nf_ref, we1a_ref, we1b_ref, wn1a_ref, ps_ref, pr_ref, qn_ref):
    nf = nf_ref[...]
    ps_ref[...] = jnp.dot(nf, we1a_ref[...], preferred_element_type=jnp.float32)
    pr_ref[...] = jnp.dot(nf, we1b_ref[...], preferred_element_type=jnp.float32)
    qn_ref[...] = jnp.dot(nf, wn1a_ref[...], preferred_element_type=jnp.float32)


def _pre(nf, we1a, we1b, wn1a):
    out = jax.ShapeDtypeStruct((N, D), jnp.float32)
    return pl.pallas_call(
        _pre_body,
        out_shape=(out, out, out),
    )(nf, we1a, we1b, wn1a)


def _gather_body(ps_hbm, pr_hbm, s_hbm, r_hbm, out_hbm,
                 sidx_all, ridx_all, a0, a1, b0, b1, c0, c1,
                 gsem0, gsem1, ssem0, ssem1):
    wid = lax.axis_index("c") * NS + lax.axis_index("s")
    base0 = wid * EPW
    abuf = (a0, a1)
    bbuf = (b0, b1)
    cbuf = (c0, c1)
    gsem = (gsem0, gsem1)
    ssem = (ssem0, ssem1)

    pltpu.sync_copy(s_hbm.at[wid], sidx_all)
    pltpu.sync_copy(r_hbm.at[wid], ridx_all)

    def fire_g(k, b):
        sl = pl.ds(k * GC, GC)
        pltpu.async_copy(ps_hbm.at[sidx_all.at[sl]], abuf[b], gsem[b])
        pltpu.async_copy(pr_hbm.at[ridx_all.at[sl]], bbuf[b], gsem[b])

    def wait_g(b):
        pltpu.make_async_copy(ps_hbm.at[pl.ds(0, GC)], abuf[b], gsem[b]).wait()
        pltpu.make_async_copy(ps_hbm.at[pl.ds(0, GC)], bbuf[b], gsem[b]).wait()

    def vadd(b):
        def row(i, carry):
            for j in range(D // 16):
                sl16 = pl.ds(j * 16, 16)
                cbuf[b][i, sl16] = abuf[b][i, sl16] + bbuf[b][i, sl16]
            return carry
        lax.fori_loop(0, GC, row, 0, unroll=4)

    def fire_s(k, b):
        pltpu.async_copy(cbuf[b], out_hbm.at[pl.ds(base0 + k * GC, GC)], ssem[b])

    def wait_s(b):
        pltpu.make_async_copy(cbuf[b], out_hbm.at[pl.ds(base0, GC)], ssem[b]).wait()

    fire_g(0, 0)
    fire_g(1, 1)

    def pair(i, carry):
        for b in (0, 1):
            k = 2 * i + b
            wait_g(b)

            @pl.when(k >= 2)
            def _():
                wait_s(b)

            vadd(b)
            fire_s(k, b)

            @pl.when(k + 2 <= NCHUNK - 1)
            def _():
                fire_g(k + 2, b)
        return carry

    lax.fori_loop(0, NCHUNK // 2, pair, 0)

    wait_g(0)
    wait_s(0)
    vadd(0)
    fire_s(NCHUNK - 1, 0)
    wait_s(1)
    wait_s(0)


@functools.lru_cache(maxsize=None)
def _gather_kernel():
    buf = pltpu.VMEM((GC, D), jnp.float32)
    return pl.kernel(
        _gather_body,
        out_type=jax.ShapeDtypeStruct((E, D), jnp.float32),
        mesh=_mesh(),
        scratch_types=[
            pltpu.VMEM((EPW,), jnp.int32),
            pltpu.VMEM((EPW,), jnp.int32),
            buf, buf, buf, buf, buf, buf,
            pltpu.SemaphoreType.DMA,
            pltpu.SemaphoreType.DMA,
            pltpu.SemaphoreType.DMA,
            pltpu.SemaphoreType.DMA,
        ],
    )


def _edge_body(g_ref, ef_ref, we1c_ref, be1_ref, we2_ref, be2_ref,
               ge_ref, bbe_ref, oe_ref, ne_ref):
    ef = ef_ref[...]
    x = g_ref[...] + jnp.dot(ef, we1c_ref[...],
                             preferred_element_type=jnp.float32) + be1_ref[...]
    h = jnp.maximum(x, 0.0)
    h2 = jnp.dot(h, we2_ref[...], preferred_element_type=jnp.float32) + be2_ref[...]
    mu = jnp.mean(h2, axis=-1, keepdims=True)
    var = jnp.mean((h2 - mu) * (h2 - mu), axis=-1, keepdims=True)
    ne = (h2 - mu) * lax.rsqrt(var + 1e-5) * ge_ref[...] + bbe_ref[...]
    ne_ref[...] = ne
    oe_ref[...] = ef + ne


def _edge_mlp(g, ef, we1c, be1, we2, be2, ge, bbe):
    BE = 2000
    grid = E // BE
    row_spec = pl.BlockSpec((BE, D), lambda i: (i, 0))
    w_spec = pl.BlockSpec((D, H), lambda i: (0, 0))
    v_spec = pl.BlockSpec((1, D), lambda i: (0, 0))
    out = jax.ShapeDtypeStruct((E, D), jnp.float32)
    return pl.pallas_call(
        _edge_body,
        grid=(grid,),
        in_specs=[row_spec, row_spec, w_spec, v_spec, w_spec, v_spec, v_spec, v_spec],
        out_specs=(row_spec, row_spec),
        out_shape=(out, out),
    )(g, ef, we1c, be1, we2, be2, ge, bbe)


def _segsum_body(ne_hbm, r_hbm, zeros_hbm, out_hbm,
                 ridx, ld0, ld1, agg, lsem0, lsem1, ssem0, ssem1):
    cid = lax.axis_index("c")
    sid = lax.axis_index("s")
    ld = (ld0, ld1)
    lsem = (lsem0, lsem1)
    ssem = (ssem0, ssem1)

    base0 = (cid * NS + sid) * EPW
    wid = cid * NS + sid

    pltpu.sync_copy(r_hbm.at[wid], ridx)

    @pl.when(sid == 0)
    def _():
        pltpu.sync_copy(zeros_hbm, agg)

    plsc.subcore_barrier()

    def fire_l(k, b):
        pltpu.async_copy(ne_hbm.at[pl.ds(base0 + k * GC, GC)], ld[b], lsem[b])

    def wait_l(b):
        pltpu.make_async_copy(ne_hbm.at[pl.ds(base0, GC)], ld[b], lsem[b]).wait()

    def fire_sc(k, b):
        pltpu.async_copy(ld[b], agg.at[ridx.at[k]], ssem[b], add=True)

    def wait_sc(b):
        pltpu.make_async_copy(ld[b], agg.at[ridx.at[0]], ssem[b]).wait()

    fire_l(0, 0)

    def pair(i, carry):
        for b in (0, 1):
            k = 2 * i + b

            @pl.when(k >= 1)
            def _():
                wait_sc(1 - b)

            @pl.when(k + 1 <= NCHUNK - 1)
            def _():
                fire_l(k + 1, 1 - b)

            wait_l(b)
            fire_sc(k, b)
        return carry

    lax.fori_loop(0, NCHUNK // 2, pair, 0)

    wait_sc(1)
    wait_l(0)
    fire_sc(NCHUNK - 1, 0)
    wait_sc(0)

    plsc.subcore_barrier()

    rows = 624
    sl = pl.ds(sid * rows, rows)
    pltpu.sync_copy(agg.at[sl], out_hbm.at[cid].at[sl])

    @pl.when(sid == 0)
    def _():
        tail = pl.ds(NS * rows, N - NS * rows)
        pltpu.sync_copy(agg.at[tail], out_hbm.at[cid].at[tail])


@functools.lru_cache(maxsize=None)
def _segsum_kernel():
    return pl.kernel(
        _segsum_body,
        out_type=jax.ShapeDtypeStruct((NC, N, D), jnp.float32),
        mesh=_mesh(),
        scratch_types=[
            pltpu.VMEM((NCHUNK, GC), jnp.int32),
            pltpu.VMEM((GC, D), jnp.float32),
            pltpu.VMEM((GC, D), jnp.float32),
            pltpu.VMEM_SHARED((N, D), jnp.float32),
            pltpu.SemaphoreType.DMA,
            pltpu.SemaphoreType.DMA,
            pltpu.SemaphoreType.DMA,
            pltpu.SemaphoreType.DMA,
        ],
    )


def _node_body(nf_ref, qn_ref, p0_ref, p1_ref, wn1b_ref, bn1_ref,
               wn2_ref, bn2_ref, gn_ref, bbn_ref, on_ref):
    agg = p0_ref[...] + p1_ref[...]
    x = qn_ref[...] + jnp.dot(agg, wn1b_ref[...],
                              preferred_element_type=jnp.float32) + bn1_ref[...]
    h = jnp.maximum(x, 0.0)
    h2 = jnp.dot(h, wn2_ref[...], preferred_element_type=jnp.float32) + bn2_ref[...]
    mu = jnp.mean(h2, axis=-1, keepdims=True)
    var = jnp.mean((h2 - mu) * (h2 - mu), axis=-1, keepdims=True)
    nn = (h2 - mu) * lax.rsqrt(var + 1e-5) * gn_ref[...] + bbn_ref[...]
    on_ref[...] = nf_ref[...] + nn


def _node_mlp(nf, qn, p0, p1, wn1b, bn1, wn2, bn2, gn, bbn):
    BN = 2000
    grid = N // BN
    row_spec = pl.BlockSpec((BN, D), lambda i: (i, 0))
    w_spec = pl.BlockSpec((D, H), lambda i: (0, 0))
    v_spec = pl.BlockSpec((1, D), lambda i: (0, 0))
    return pl.pallas_call(
        _node_body,
        grid=(grid,),
        in_specs=[row_spec, row_spec, row_spec, row_spec,
                  w_spec, v_spec, w_spec, v_spec, v_spec, v_spec],
        out_specs=row_spec,
        out_shape=jax.ShapeDtypeStruct((N, D), jnp.float32),
    )(nf, qn, p0, p1, wn1b, bn1, wn2, bn2, gn, bbn)


def kernel(node_features, senders, receivers, edge_features,
           We1, be1, We2, be2, ge, bbe, Wn1, bn1, Wn2, bn2, gn, bbn):
    senders = senders.astype(jnp.int32)
    receivers = receivers.astype(jnp.int32)

    we1a, we1b, we1c = We1[0:D], We1[D:2 * D], We1[2 * D:3 * D]
    wn1a, wn1b = Wn1[0:D], Wn1[D:2 * D]

    s2 = senders.reshape(NW, EPW)
    r2 = receivers.reshape(NW, EPW)
    r3 = receivers.reshape(NW, NCHUNK, GC)

    ps, pr, qn = _pre(node_features, we1a, we1b, wn1a)
    g = _gather_kernel()(ps, pr, s2, r2)
    out_edges, new_edge = _edge_mlp(
        g, edge_features, we1c,
        be1.reshape(1, D), We2, be2.reshape(1, D),
        ge.reshape(1, D), bbe.reshape(1, D))
    zeros = jnp.zeros((N, D), jnp.float32)
    parts = _segsum_kernel()(new_edge, r3, zeros)
    out_nodes = _node_mlp(
        node_features, qn, parts[0], parts[1],
        wn1b, bn1.reshape(1, D), Wn2, bn2.reshape(1, D),
        gn.reshape(1, D), bbn.reshape(1, D))
    return (out_nodes, out_edges)
